# baseline (speedup 1.0000x reference)
import functools

import jax
import jax.numpy as jnp
from jax.experimental import pallas as pl
from jax.experimental.pallas import tpu as pltpu

_CHUNK = 2048


def _linsum(v):
    acc = v[:, 0:1]
    for d in range(1, v.shape[1]):
        acc = acc + v[:, d:d + 1]
    return acc


def _vq_kernel(x_ref, cb_ref, w_ref,
               outq_ref, idx_ref, err_ref,
               impT_s, cs_s, res_s,
               *, tile_t: int, K: int):
    t = pl.program_id(0)
    q = pl.program_id(1)

    @pl.when(t == 0)
    def _():
        impT = jax.lax.dot_general(
            w_ref[q].astype(jnp.bfloat16), cb_ref[q].astype(jnp.bfloat16),
            dimension_numbers=(((1,), (1,)), ((), ())),
            preferred_element_type=jnp.float32)
        impT_s[q] = impT
        sq = impT * impT
        cs = sq[0:1, :]
        for dd in range(1, sq.shape[0]):
            cs = cs + sq[dd:dd + 1, :]
        cs_s[q] = cs

    @pl.when(q == 0)
    def _():
        res_s[...] = x_ref[...]

    s = res_s[...]
    impT = impT_s[q]

    rc = jax.lax.dot_general(
        s.astype(jnp.bfloat16), impT.astype(jnp.bfloat16),
        dimension_numbers=(((1,), (0,)), ((), ())),
        preferred_element_type=jnp.float32)
    xs = _linsum(s * s)
    d = (xs + cs_s[q]) - 2.0 * rc

    nch = K // _CHUNK
    iota = jax.lax.broadcasted_iota(jnp.int32, (tile_t, _CHUNK), 1)
    mvs, mis = [], []
    for j in range(nch):
        blk = d[:, j * _CHUNK:(j + 1) * _CHUNK]
        mv = jnp.min(blk, axis=1, keepdims=True)
        mi = jnp.min(jnp.where(blk == mv, iota + j * _CHUNK, K),
                     axis=1, keepdims=True)
        mvs.append(mv)
        mis.append(mi)

    gmin = mvs[0]
    for j in range(1, nch):
        gmin = jnp.minimum(gmin, mvs[j])
    idx_plain = jnp.full_like(mis[0], K)
    for j in range(nch):
        idx_plain = jnp.minimum(idx_plain,
                                jnp.where(mvs[j] == gmin, mis[j], K))

    acc = jnp.full((tile_t, 1), jnp.inf, jnp.float32)
    idx_carry = jnp.zeros_like(mis[0])
    for j in range(nch):
        take = mvs[j] < acc
        acc = jnp.where(take, mvs[j].astype(jnp.bfloat16).astype(jnp.float32),
                        acc)
        idx_carry = jnp.where(take, mis[j], idx_carry)

    idx = jnp.where(q == 0, idx_plain, idx_carry)

    iota_k = jax.lax.broadcasted_iota(jnp.int32, (tile_t, K), 1)
    oh = (iota_k == idx).astype(jnp.float32)
    tq = jax.lax.dot_general(
        oh, impT, dimension_numbers=(((1,), (1,)), ((), ())),
        precision=jax.lax.Precision.HIGHEST,
        preferred_element_type=jnp.float32)

    diff = s - tq + 1e-6
    err_ref[...] = _linsum(diff * diff)[None]

    norm_s = jnp.sqrt(xs)
    norm_t = jnp.sqrt(_linsum(tq * tq))
    u = s / jnp.maximum(norm_s, 1e-12)
    qv = tq / jnp.maximum(norm_t, 1e-12)
    wsum = u + qv
    wn = jnp.sqrt(_linsum(wsum * wsum))
    w = wsum / jnp.maximum(wn, 1e-12)
    sw = _linsum(s * w)
    su = _linsum(s * u)
    rot = (s - 2.0 * sw * w) + 2.0 * su * qv
    scale = norm_t / jnp.maximum(norm_s, 1e-6)
    rotq = rot * scale

    res_s[...] = s - rotq

    @pl.when(q == 0)
    def _():
        outq_ref[...] = rotq

    @pl.when(q > 0)
    def _():
        outq_ref[...] = outq_ref[...] + rotq

    idx_ref[...] = idx[None]


def kernel(x, frozen_codebooks, W):
    B, N, D = x.shape
    Q, K, _ = frozen_codebooks.shape
    T = B * N
    TILE_T = 256
    NT = T // TILE_T

    x2 = x.reshape(T, D)

    grid = (NT, Q)
    outq, idx, err = pl.pallas_call(
        functools.partial(_vq_kernel, tile_t=TILE_T, K=K),
        grid=grid,
        in_specs=[
            pl.BlockSpec((TILE_T, D), lambda t, q: (t, 0)),
            pl.BlockSpec((Q, K, D), lambda t, q: (0, 0, 0)),
            pl.BlockSpec((Q, D, D), lambda t, q: (0, 0, 0)),
        ],
        out_specs=[
            pl.BlockSpec((TILE_T, D), lambda t, q: (t, 0)),
            pl.BlockSpec((1, TILE_T, 1), lambda t, q: (q, t, 0)),
            pl.BlockSpec((1, TILE_T, 1), lambda t, q: (q, t, 0)),
        ],
        out_shape=[
            jax.ShapeDtypeStruct((T, D), jnp.float32),
            jax.ShapeDtypeStruct((Q, T, 1), jnp.int32),
            jax.ShapeDtypeStruct((Q, T, 1), jnp.float32),
        ],
        scratch_shapes=[
            pltpu.VMEM((Q, D, K), jnp.float32),
            pltpu.VMEM((Q, 1, K), jnp.float32),
            pltpu.VMEM((TILE_T, D), jnp.float32),
        ],
    )(x2, frozen_codebooks, W)

    quantized_out = outq.reshape(B, N, D)
    all_indices = jnp.transpose(idx.reshape(Q, B, N), (1, 2, 0))
    all_losses = jnp.mean(err.reshape(Q, T), axis=1)
    return quantized_out, all_indices, all_losses


# speedup vs baseline: 2.0726x; 2.0726x over previous
import functools

import jax
import jax.numpy as jnp
from jax.experimental import pallas as pl
from jax.experimental.pallas import tpu as pltpu

_CHUNK = 2048


def _vq_kernel(x_ref, cb_ref, w_ref,
               outq_ref, idx_ref, err_ref,
               impT_s, cs_s, res_s,
               *, tile_t: int, K: int):
    t = pl.program_id(0)
    q = pl.program_id(1)

    @pl.when(t == 0)
    def _():
        impT = jax.lax.dot_general(
            w_ref[q].astype(jnp.bfloat16), cb_ref[q].astype(jnp.bfloat16),
            dimension_numbers=(((1,), (1,)), ((), ())),
            preferred_element_type=jnp.float32)
        impT_s[q] = impT
        cs_s[q] = jnp.sum(impT * impT, axis=0, keepdims=True)

    @pl.when(q == 0)
    def _():
        res_s[...] = x_ref[...]

    s = res_s[...]
    impT = impT_s[q]

    rc = jax.lax.dot_general(
        s.astype(jnp.bfloat16), impT.astype(jnp.bfloat16),
        dimension_numbers=(((1,), (0,)), ((), ())),
        preferred_element_type=jnp.float32)
    xs = jnp.sum(s * s, axis=1, keepdims=True)
    d = (xs + cs_s[q]) - 2.0 * rc

    nch = K // _CHUNK
    iota = jax.lax.broadcasted_iota(jnp.int32, (tile_t, _CHUNK), 1)
    mvs, mis = [], []
    for j in range(nch):
        blk = d[:, j * _CHUNK:(j + 1) * _CHUNK]
        mv = jnp.min(blk, axis=1, keepdims=True)
        mi = jnp.min(jnp.where(blk == mv, iota + j * _CHUNK, K),
                     axis=1, keepdims=True)
        mvs.append(mv)
        mis.append(mi)

    gmin = mvs[0]
    for j in range(1, nch):
        gmin = jnp.minimum(gmin, mvs[j])
    idx_plain = jnp.full_like(mis[0], K)
    for j in range(nch):
        idx_plain = jnp.minimum(idx_plain,
                                jnp.where(mvs[j] == gmin, mis[j], K))

    acc = jnp.full((tile_t, 1), jnp.inf, jnp.float32)
    idx_carry = jnp.zeros_like(mis[0])
    for j in range(nch):
        take = mvs[j] < acc
        acc = jnp.where(take, mvs[j].astype(jnp.bfloat16).astype(jnp.float32),
                        acc)
        idx_carry = jnp.where(take, mis[j], idx_carry)

    idx = jnp.where(q == 0, idx_plain, idx_carry)

    iota_k = jax.lax.broadcasted_iota(jnp.int32, (tile_t, K), 1)
    oh = (iota_k == idx).astype(jnp.float32)
    tq = jax.lax.dot_general(
        oh, impT, dimension_numbers=(((1,), (1,)), ((), ())),
        precision=jax.lax.Precision.HIGHEST,
        preferred_element_type=jnp.float32)

    diff = s - tq + 1e-6
    err_ref[...] = jnp.sum(diff * diff, axis=1, keepdims=True)[None]

    norm_s = jnp.sqrt(xs)
    norm_t = jnp.sqrt(jnp.sum(tq * tq, axis=1, keepdims=True))
    u = s / jnp.maximum(norm_s, 1e-12)
    qv = tq / jnp.maximum(norm_t, 1e-12)
    wsum = u + qv
    wn = jnp.sqrt(jnp.sum(wsum * wsum, axis=1, keepdims=True))
    w = wsum / jnp.maximum(wn, 1e-12)
    sw = jnp.sum(s * w, axis=1, keepdims=True)
    su = jnp.sum(s * u, axis=1, keepdims=True)
    rot = (s - 2.0 * sw * w) + 2.0 * su * qv
    scale = norm_t / jnp.maximum(norm_s, 1e-6)
    rotq = rot * scale

    res_s[...] = s - rotq

    @pl.when(q == 0)
    def _():
        outq_ref[...] = rotq

    @pl.when(q > 0)
    def _():
        outq_ref[...] = outq_ref[...] + rotq

    idx_ref[...] = idx[None]


def kernel(x, frozen_codebooks, W):
    B, N, D = x.shape
    Q, K, _ = frozen_codebooks.shape
    T = B * N
    TILE_T = 512
    NT = T // TILE_T

    x2 = x.reshape(T, D)

    grid = (NT, Q)
    outq, idx, err = pl.pallas_call(
        functools.partial(_vq_kernel, tile_t=TILE_T, K=K),
        grid=grid,
        in_specs=[
            pl.BlockSpec((TILE_T, D), lambda t, q: (t, 0)),
            pl.BlockSpec((Q, K, D), lambda t, q: (0, 0, 0)),
            pl.BlockSpec((Q, D, D), lambda t, q: (0, 0, 0)),
        ],
        out_specs=[
            pl.BlockSpec((TILE_T, D), lambda t, q: (t, 0)),
            pl.BlockSpec((1, TILE_T, 1), lambda t, q: (q, t, 0)),
            pl.BlockSpec((1, TILE_T, 1), lambda t, q: (q, t, 0)),
        ],
        out_shape=[
            jax.ShapeDtypeStruct((T, D), jnp.float32),
            jax.ShapeDtypeStruct((Q, T, 1), jnp.int32),
            jax.ShapeDtypeStruct((Q, T, 1), jnp.float32),
        ],
        scratch_shapes=[
            pltpu.VMEM((Q, D, K), jnp.float32),
            pltpu.VMEM((Q, 1, K), jnp.float32),
            pltpu.VMEM((TILE_T, D), jnp.float32),
        ],
    )(x2, frozen_codebooks, W)

    quantized_out = outq.reshape(B, N, D)
    all_indices = jnp.transpose(idx.reshape(Q, B, N), (1, 2, 0))
    all_losses = jnp.mean(err.reshape(Q, T), axis=1)
    return quantized_out, all_indices, all_losses


# speedup vs baseline: 2.8837x; 1.3913x over previous
import functools

import jax
import jax.numpy as jnp
from jax.experimental import pallas as pl
from jax.experimental.pallas import tpu as pltpu

_CHUNK = 2048


def _vq_kernel(x_ref, cb_ref, w_ref,
               outq_ref, idx_ref, err_ref,
               impT_s, cs_s, res_s,
               *, tile_t: int, K: int):
    t = pl.program_id(0)
    q = pl.program_id(1)

    @pl.when(t == 0)
    def _():
        impTq = jax.lax.dot_general(
            w_ref[q].astype(jnp.bfloat16), cb_ref[q].astype(jnp.bfloat16),
            dimension_numbers=(((1,), (1,)), ((), ())),
            preferred_element_type=jnp.float32)
        cs_s[q] = jnp.sum(impTq * impTq, axis=0, keepdims=True)
        hi = impTq.astype(jnp.bfloat16)
        rem1 = impTq - hi.astype(jnp.float32)
        mid = rem1.astype(jnp.bfloat16)
        lo = (rem1 - mid.astype(jnp.float32)).astype(jnp.bfloat16)
        impT_s[q, 0] = hi
        impT_s[q, 1] = mid
        impT_s[q, 2] = lo

    @pl.when(q == 0)
    def _():
        res_s[...] = x_ref[...]

    s = res_s[...]
    impT_hi = impT_s[q, 0]

    rc = jax.lax.dot_general(
        s.astype(jnp.bfloat16), impT_hi,
        dimension_numbers=(((1,), (0,)), ((), ())),
        preferred_element_type=jnp.float32)
    xs = jnp.sum(s * s, axis=1, keepdims=True)
    d = (xs + cs_s[q]) - 2.0 * rc

    nch = K // _CHUNK
    iota = jax.lax.broadcasted_iota(jnp.int32, (tile_t, _CHUNK), 1)
    mvs, mis = [], []
    for j in range(nch):
        blk = d[:, j * _CHUNK:(j + 1) * _CHUNK]
        mv = jnp.min(blk, axis=1, keepdims=True)
        mi = jnp.min(jnp.where(blk == mv, iota + j * _CHUNK, K),
                     axis=1, keepdims=True)
        mvs.append(mv)
        mis.append(mi)

    gmin = mvs[0]
    for j in range(1, nch):
        gmin = jnp.minimum(gmin, mvs[j])
    idx_plain = jnp.full_like(mis[0], K)
    for j in range(nch):
        idx_plain = jnp.minimum(idx_plain,
                                jnp.where(mvs[j] == gmin, mis[j], K))

    acc = jnp.full((tile_t, 1), jnp.inf, jnp.float32)
    idx_carry = jnp.zeros_like(mis[0])
    for j in range(nch):
        take = mvs[j] < acc
        acc = jnp.where(take, mvs[j].astype(jnp.bfloat16).astype(jnp.float32),
                        acc)
        idx_carry = jnp.where(take, mis[j], idx_carry)

    idx = jnp.where(q == 0, idx_plain, idx_carry)

    iota_k = jax.lax.broadcasted_iota(jnp.int32, (tile_t, K), 1)
    oh = (iota_k == idx).astype(jnp.bfloat16)
    parts = []
    for p in range(3):
        parts.append(jax.lax.dot_general(
            oh, impT_s[q, p], dimension_numbers=(((1,), (1,)), ((), ())),
            preferred_element_type=jnp.float32))
    tq = (parts[0] + parts[1]) + parts[2]

    diff = s - tq + 1e-6
    err_ref[...] = jnp.sum(diff * diff, axis=1, keepdims=True)[None]

    norm_s = jnp.sqrt(xs)
    norm_t = jnp.sqrt(jnp.sum(tq * tq, axis=1, keepdims=True))
    u = s / jnp.maximum(norm_s, 1e-12)
    qv = tq / jnp.maximum(norm_t, 1e-12)
    wsum = u + qv
    wn = jnp.sqrt(jnp.sum(wsum * wsum, axis=1, keepdims=True))
    w = wsum / jnp.maximum(wn, 1e-12)
    sw = jnp.sum(s * w, axis=1, keepdims=True)
    su = jnp.sum(s * u, axis=1, keepdims=True)
    rot = (s - 2.0 * sw * w) + 2.0 * su * qv
    scale = norm_t / jnp.maximum(norm_s, 1e-6)
    rotq = rot * scale

    res_s[...] = s - rotq

    @pl.when(q == 0)
    def _():
        outq_ref[...] = rotq

    @pl.when(q > 0)
    def _():
        outq_ref[...] = outq_ref[...] + rotq

    idx_ref[...] = idx[None]


def kernel(x, frozen_codebooks, W):
    B, N, D = x.shape
    Q, K, _ = frozen_codebooks.shape
    T = B * N
    TILE_T = 512
    NT = T // TILE_T

    x2 = x.reshape(T, D)

    grid = (NT, Q)
    outq, idx, err = pl.pallas_call(
        functools.partial(_vq_kernel, tile_t=TILE_T, K=K),
        grid=grid,
        in_specs=[
            pl.BlockSpec((TILE_T, D), lambda t, q: (t, 0)),
            pl.BlockSpec((Q, K, D), lambda t, q: (0, 0, 0)),
            pl.BlockSpec((Q, D, D), lambda t, q: (0, 0, 0)),
        ],
        out_specs=[
            pl.BlockSpec((TILE_T, D), lambda t, q: (t, 0)),
            pl.BlockSpec((1, TILE_T, 1), lambda t, q: (q, t, 0)),
            pl.BlockSpec((1, TILE_T, 1), lambda t, q: (q, t, 0)),
        ],
        out_shape=[
            jax.ShapeDtypeStruct((T, D), jnp.float32),
            jax.ShapeDtypeStruct((Q, T, 1), jnp.int32),
            jax.ShapeDtypeStruct((Q, T, 1), jnp.float32),
        ],
        scratch_shapes=[
            pltpu.VMEM((Q, 3, D, K), jnp.bfloat16),
            pltpu.VMEM((Q, 1, K), jnp.float32),
            pltpu.VMEM((TILE_T, D), jnp.float32),
        ],
    )(x2, frozen_codebooks, W)

    quantized_out = outq.reshape(B, N, D)
    all_indices = jnp.transpose(idx.reshape(Q, B, N), (1, 2, 0))
    all_losses = jnp.mean(err.reshape(Q, T), axis=1)
    return quantized_out, all_indices, all_losses


# speedup vs baseline: 2.9586x; 1.0260x over previous
import functools

import jax
import jax.numpy as jnp
from jax.experimental import pallas as pl
from jax.experimental.pallas import tpu as pltpu

_CHUNK = 2048


def _vq_kernel(x_ref, cb_ref, w_ref,
               outq_ref, idx_ref, err_ref,
               impT_s, cs_s, res_s,
               *, tile_t: int, K: int):
    t = pl.program_id(0)
    q = pl.program_id(1)

    @pl.when(t == 0)
    def _():
        impTq = jax.lax.dot_general(
            w_ref[q].astype(jnp.bfloat16), cb_ref[q].astype(jnp.bfloat16),
            dimension_numbers=(((1,), (1,)), ((), ())),
            preferred_element_type=jnp.float32)
        cs_s[q] = jnp.sum(impTq * impTq, axis=0, keepdims=True)
        hi = impTq.astype(jnp.bfloat16)
        rem1 = impTq - hi.astype(jnp.float32)
        mid = rem1.astype(jnp.bfloat16)
        lo = (rem1 - mid.astype(jnp.float32)).astype(jnp.bfloat16)
        impT_s[q, 0] = hi
        impT_s[q, 1] = mid
        impT_s[q, 2] = lo

    @pl.when(q == 0)
    def _():
        res_s[...] = x_ref[...]

    s = res_s[...]
    impT_hi = impT_s[q, 0]

    rc = jax.lax.dot_general(
        s.astype(jnp.bfloat16), impT_hi,
        dimension_numbers=(((1,), (0,)), ((), ())),
        preferred_element_type=jnp.float32)
    xs = jnp.sum(s * s, axis=1, keepdims=True)
    d = (xs + cs_s[q]) - 2.0 * rc

    nch = K // _CHUNK
    blks, mvs = [], []
    for j in range(nch):
        blk = d[:, j * _CHUNK:(j + 1) * _CHUNK]
        blks.append(blk)
        mvs.append(jnp.min(blk, axis=1, keepdims=True))

    gmin = mvs[0]
    for j in range(1, nch):
        gmin = jnp.minimum(gmin, mvs[j])
    jw_plain = jnp.full((tile_t, 1), nch, jnp.int32)
    for j in range(nch):
        jw_plain = jnp.minimum(jw_plain,
                               jnp.where(mvs[j] == gmin, j, nch))

    acc = jnp.full((tile_t, 1), jnp.inf, jnp.float32)
    jw_carry = jnp.zeros((tile_t, 1), jnp.int32)
    for j in range(nch):
        take = mvs[j] < acc
        acc = jnp.where(take, mvs[j].astype(jnp.bfloat16).astype(jnp.float32),
                        acc)
        jw_carry = jnp.where(take, j, jw_carry)

    jw = jnp.where(q == 0, jw_plain, jw_carry)

    blk_win = blks[nch - 1]
    mv_win = mvs[nch - 1]
    for j in range(nch - 2, -1, -1):
        sel = jw == j
        blk_win = jnp.where(sel, blks[j], blk_win)
        mv_win = jnp.where(sel, mvs[j], mv_win)
    iota = jax.lax.broadcasted_iota(jnp.int32, (tile_t, _CHUNK), 1)
    idx = (jnp.min(jnp.where(blk_win == mv_win, iota, _CHUNK),
                   axis=1, keepdims=True) + jw * _CHUNK)

    iota_k = jax.lax.broadcasted_iota(jnp.int32, (tile_t, K), 1)
    oh = (iota_k == idx).astype(jnp.bfloat16)
    parts = []
    for p in range(3):
        parts.append(jax.lax.dot_general(
            oh, impT_s[q, p], dimension_numbers=(((1,), (1,)), ((), ())),
            preferred_element_type=jnp.float32))
    tq = (parts[0] + parts[1]) + parts[2]

    diff = s - tq + 1e-6
    err_ref[...] = jnp.sum(diff * diff, axis=1, keepdims=True)[None]

    norm_s = jnp.sqrt(xs)
    norm_t = jnp.sqrt(jnp.sum(tq * tq, axis=1, keepdims=True))
    u = s / jnp.maximum(norm_s, 1e-12)
    qv = tq / jnp.maximum(norm_t, 1e-12)
    wsum = u + qv
    wn = jnp.sqrt(jnp.sum(wsum * wsum, axis=1, keepdims=True))
    w = wsum / jnp.maximum(wn, 1e-12)
    sw = jnp.sum(s * w, axis=1, keepdims=True)
    su = jnp.sum(s * u, axis=1, keepdims=True)
    rot = (s - 2.0 * sw * w) + 2.0 * su * qv
    scale = norm_t / jnp.maximum(norm_s, 1e-6)
    rotq = rot * scale

    res_s[...] = s - rotq

    @pl.when(q == 0)
    def _():
        outq_ref[...] = rotq

    @pl.when(q > 0)
    def _():
        outq_ref[...] = outq_ref[...] + rotq

    idx_ref[...] = idx[None]


def kernel(x, frozen_codebooks, W):
    B, N, D = x.shape
    Q, K, _ = frozen_codebooks.shape
    T = B * N
    TILE_T = 512
    NT = T // TILE_T

    x2 = x.reshape(T, D)

    grid = (NT, Q)
    outq, idx, err = pl.pallas_call(
        functools.partial(_vq_kernel, tile_t=TILE_T, K=K),
        grid=grid,
        in_specs=[
            pl.BlockSpec((TILE_T, D), lambda t, q: (t, 0)),
            pl.BlockSpec((Q, K, D), lambda t, q: (0, 0, 0)),
            pl.BlockSpec((Q, D, D), lambda t, q: (0, 0, 0)),
        ],
        out_specs=[
            pl.BlockSpec((TILE_T, D), lambda t, q: (t, 0)),
            pl.BlockSpec((1, TILE_T, 1), lambda t, q: (q, t, 0)),
            pl.BlockSpec((1, TILE_T, 1), lambda t, q: (q, t, 0)),
        ],
        out_shape=[
            jax.ShapeDtypeStruct((T, D), jnp.float32),
            jax.ShapeDtypeStruct((Q, T, 1), jnp.int32),
            jax.ShapeDtypeStruct((Q, T, 1), jnp.float32),
        ],
        scratch_shapes=[
            pltpu.VMEM((Q, 3, D, K), jnp.bfloat16),
            pltpu.VMEM((Q, 1, K), jnp.float32),
            pltpu.VMEM((TILE_T, D), jnp.float32),
        ],
    )(x2, frozen_codebooks, W)

    quantized_out = outq.reshape(B, N, D)
    all_indices = jnp.transpose(idx.reshape(Q, B, N), (1, 2, 0))
    all_losses = jnp.mean(err.reshape(Q, T), axis=1)
    return quantized_out, all_indices, all_losses
